# initial kernel scaffold (unmeasured)
import jax
import jax.numpy as jnp
from jax import lax
from jax.experimental import pallas as pl
from jax.experimental.pallas import tpu as pltpu


def kernel(
    x,
):
    def body(*refs):
        pass

    out_shape = jax.ShapeDtypeStruct(..., jnp.float32)
    return pl.pallas_call(body, out_shape=out_shape)(...)


# baseline (device time: 13449 ns/iter reference)
import jax
import jax.numpy as jnp
from jax import lax
from jax.experimental import pallas as pl
from jax.experimental.pallas import tpu as pltpu

N_DEV = 32


def kernel(x):
    m, n = x.shape

    def body(x_ref, out_ref, halo_prev, halo_next, send_sems, recv_sems):
        my_i = lax.axis_index("i")
        has_left = my_i > 0
        has_right = my_i < N_DEV - 1

        halo_prev[0, :] = jnp.zeros((n,), jnp.float32)
        halo_next[0, :] = jnp.zeros((n,), jnp.float32)

        barrier_sem = pltpu.get_barrier_semaphore()

        @pl.when(has_left)
        def _():
            pl.semaphore_signal(
                barrier_sem, inc=1,
                device_id=(my_i - 1,), device_id_type=pl.DeviceIdType.MESH,
            )

        @pl.when(has_right)
        def _():
            pl.semaphore_signal(
                barrier_sem, inc=1,
                device_id=(my_i + 1,), device_id_type=pl.DeviceIdType.MESH,
            )

        @pl.when(has_left)
        def _():
            pl.semaphore_wait(barrier_sem, 1)

        @pl.when(has_right)
        def _():
            pl.semaphore_wait(barrier_sem, 1)

        @pl.when(has_right)
        def _():
            rdma = pltpu.make_async_remote_copy(
                src_ref=x_ref.at[pl.ds(m - 1, 1), :],
                dst_ref=halo_prev,
                send_sem=send_sems.at[0],
                recv_sem=recv_sems.at[0],
                device_id=(my_i + 1,),
                device_id_type=pl.DeviceIdType.MESH,
            )
            rdma.start()
            rdma.wait_send()

        @pl.when(has_left)
        def _():
            rdma = pltpu.make_async_remote_copy(
                src_ref=x_ref.at[pl.ds(0, 1), :],
                dst_ref=halo_next,
                send_sem=send_sems.at[1],
                recv_sem=recv_sems.at[1],
                device_id=(my_i - 1,),
                device_id_type=pl.DeviceIdType.MESH,
            )
            rdma.start()
            rdma.wait_send()

        x_all = x_ref[:, :]
        out_ref[pl.ds(1, m - 2), :] = (
            0.25 * x_all[: m - 2] + 0.5 * x_all[1 : m - 1] + 0.25 * x_all[2:]
        )

        @pl.when(has_left)
        def _():
            recv = pltpu.make_async_remote_copy(
                src_ref=x_ref.at[pl.ds(m - 1, 1), :],
                dst_ref=halo_prev,
                send_sem=send_sems.at[0],
                recv_sem=recv_sems.at[0],
                device_id=(my_i - 1,),
                device_id_type=pl.DeviceIdType.MESH,
            )
            recv.wait_recv()

        @pl.when(has_right)
        def _():
            recv = pltpu.make_async_remote_copy(
                src_ref=x_ref.at[pl.ds(0, 1), :],
                dst_ref=halo_next,
                send_sem=send_sems.at[1],
                recv_sem=recv_sems.at[1],
                device_id=(my_i + 1,),
                device_id_type=pl.DeviceIdType.MESH,
            )
            recv.wait_recv()

        first = x_all[0:1, :]
        second = x_all[1:2, :]
        penult = x_all[m - 2 : m - 1, :]
        last = x_all[m - 1 : m, :]
        row0 = jnp.where(
            my_i == 0,
            first,
            0.25 * halo_prev[0:1, :] + 0.5 * first + 0.25 * second,
        )
        rowm = jnp.where(
            my_i == N_DEV - 1,
            last,
            0.25 * penult + 0.5 * last + 0.25 * halo_next[0:1, :],
        )
        out_ref[pl.ds(0, 1), :] = row0
        out_ref[pl.ds(m - 1, 1), :] = rowm

    return pl.pallas_call(
        body,
        out_shape=jax.ShapeDtypeStruct((m, n), x.dtype),
        in_specs=[pl.BlockSpec(memory_space=pltpu.VMEM)],
        out_specs=pl.BlockSpec(memory_space=pltpu.VMEM),
        scratch_shapes=[
            pltpu.VMEM((1, n), jnp.float32),
            pltpu.VMEM((1, n), jnp.float32),
            pltpu.SemaphoreType.DMA((2,)),
            pltpu.SemaphoreType.DMA((2,)),
        ],
        compiler_params=pltpu.CompilerParams(collective_id=0),
    )(x)
